# baseline (speedup 1.0000x reference)
import functools

import jax
import jax.numpy as jnp
from jax import lax
from jax.experimental import pallas as pl
from jax.experimental.pallas import tpu as pltpu
from jax.experimental.pallas import tpu_sc as plsc

L = 16


def _build(B, NZ, NC, NS, R):
    NW = NC * NS
    rows_per_w = B // NW
    n_chunks = rows_per_w // R
    n_groups = NZ // L
    mesh = plsc.VectorSubcoreMesh(core_axis_name="c", subcore_axis_name="s")

    @functools.partial(
        pl.kernel,
        out_type=jax.ShapeDtypeStruct((B * NZ,), jnp.float32),
        mesh=mesh,
        compiler_params=pltpu.CompilerParams(needs_layout_passes=False),
        scratch_types=[
            pltpu.VMEM((NZ,), jnp.int32),
            pltpu.VMEM((R * NZ,), jnp.float32),
            pltpu.VMEM((R * NZ,), jnp.float32),
            pltpu.SemaphoreType.DMA,
        ],
    )
    def k(h_hbm, idx_hbm, out_hbm, idx_v, inb, outb, sem):
        wid = lax.axis_index("s") * NC + lax.axis_index("c")
        base = wid * rows_per_w * NZ
        pltpu.sync_copy(idx_hbm, idx_v)

        @pl.loop(0, n_chunks)
        def _chunk(ci):
            off = base + ci * (R * NZ)
            pltpu.async_copy(h_hbm.at[pl.ds(off, R * NZ)], inb, sem).wait()

            @pl.loop(0, n_groups)
            def _group(g):
                col = g * L
                idxg = idx_v[pl.ds(col, L)]
                for r in range(R):
                    outb[pl.ds(r * NZ + col, L)] = plsc.load_gather(
                        inb, [idxg + (r * NZ)])

            pltpu.async_copy(outb, out_hbm.at[pl.ds(off, R * NZ)], sem).wait()

    return k


def kernel(h, indices):
    B, NZ = h.shape
    info = plsc.get_sparse_core_info()
    k = _build(B, NZ, info.num_cores, info.num_subcores, R=8)
    return k(h.reshape(B * NZ), indices).reshape(B, NZ)


# speedup vs baseline: 1.2409x; 1.2409x over previous
import functools

import jax
import jax.numpy as jnp
from jax import lax
from jax.experimental import pallas as pl
from jax.experimental.pallas import tpu as pltpu
from jax.experimental.pallas import tpu_sc as plsc

L = 16


def _build(B, NZ, NC, NS, R):
    NW = NC * NS
    rows_per_w = B // NW
    n_chunks = rows_per_w // R
    n_groups = NZ // L
    C = R * NZ
    mesh = plsc.VectorSubcoreMesh(core_axis_name="c", subcore_axis_name="s")

    @functools.partial(
        pl.kernel,
        out_type=jax.ShapeDtypeStruct((B * NZ,), jnp.float32),
        mesh=mesh,
        compiler_params=pltpu.CompilerParams(needs_layout_passes=False),
        scratch_types=[
            pltpu.VMEM((NZ,), jnp.int32),
            pltpu.VMEM((C,), jnp.float32),
            pltpu.VMEM((C,), jnp.float32),
            pltpu.VMEM((C,), jnp.float32),
            pltpu.VMEM((C,), jnp.float32),
            pltpu.SemaphoreType.DMA,
            pltpu.SemaphoreType.DMA,
            pltpu.SemaphoreType.DMA,
            pltpu.SemaphoreType.DMA,
        ],
    )
    def k(h_hbm, idx_hbm, out_hbm, idx_v, in0, in1, ob0, ob1,
          si0, si1, so0, so1):
        wid = lax.axis_index("s") * NC + lax.axis_index("c")
        base = wid * rows_per_w * NZ
        pltpu.sync_copy(idx_hbm, idx_v)

        inb = (in0, in1)
        outb = (ob0, ob1)
        sin = (si0, si1)
        sout = (so0, so1)

        def start_in(ci, b):
            pltpu.async_copy(h_hbm.at[pl.ds(base + ci * C, C)], inb[b], sin[b])

        def start_out(ci, b):
            pltpu.async_copy(outb[b], out_hbm.at[pl.ds(base + ci * C, C)],
                             sout[b])

        def wait_in(b):
            pltpu.make_async_copy(
                h_hbm.at[pl.ds(base, C)], inb[b], sin[b]).wait()

        def wait_out(b):
            pltpu.make_async_copy(
                outb[b], out_hbm.at[pl.ds(base, C)], sout[b]).wait()

        def compute(b):
            src, dst = inb[b], outb[b]

            @pl.loop(0, n_groups, unroll=2)
            def _group(g):
                col = g * L
                idxg = idx_v[pl.ds(col, L)]
                for r in range(R):
                    dst[pl.ds(r * NZ + col, L)] = plsc.load_gather(
                        src, [idxg + (r * NZ)])

        start_in(0, 0)
        start_in(1, 1)

        @pl.loop(0, n_chunks, step=2)
        def _chunk(ci):
            for b in range(2):
                cb = ci + b
                wait_in(b)

                @pl.when(cb >= 2)
                def _():
                    wait_out(b)

                compute(b)
                start_out(cb, b)

                @pl.when(cb + 2 < n_chunks)
                def _():
                    start_in(cb + 2, b)

        wait_out(0)
        wait_out(1)

    return k


def kernel(h, indices):
    B, NZ = h.shape
    info = plsc.get_sparse_core_info()
    k = _build(B, NZ, info.num_cores, info.num_subcores, R=8)
    return k(h.reshape(B * NZ), indices).reshape(B, NZ)


# speedup vs baseline: 2.0421x; 1.6456x over previous
import functools

import jax
import jax.numpy as jnp
from jax import lax
from jax.experimental import pallas as pl
from jax.experimental.pallas import tpu as pltpu
from jax.experimental.pallas import tpu_sc as plsc

L = 16


def _build(B, NZ, NC, NS, R):
    NW = NC * NS
    rows_per_w = B // NW
    n_chunks = rows_per_w // R
    n_groups = NZ // L
    C = R * NZ
    mesh = plsc.VectorSubcoreMesh(core_axis_name="c", subcore_axis_name="s")

    @functools.partial(
        pl.kernel,
        out_type=jax.ShapeDtypeStruct((B * NZ,), jnp.float32),
        mesh=mesh,
        compiler_params=pltpu.CompilerParams(needs_layout_passes=False),
        scratch_types=[
            pltpu.VMEM((NZ,), jnp.int32),
            pltpu.VMEM((C,), jnp.float32),
            pltpu.VMEM((C,), jnp.float32),
            pltpu.VMEM((C,), jnp.float32),
            pltpu.VMEM((C,), jnp.float32),
            pltpu.SemaphoreType.DMA,
            pltpu.SemaphoreType.DMA,
            pltpu.SemaphoreType.DMA,
            pltpu.SemaphoreType.DMA,
        ],
    )
    def k(h_hbm, idx_hbm, out_hbm, idx_v, in0, in1, ob0, ob1,
          si0, si1, so0, so1):
        wid = lax.axis_index("s") * NC + lax.axis_index("c")
        base = wid * rows_per_w * NZ
        pltpu.sync_copy(idx_hbm, idx_v)

        inb = (in0, in1)
        outb = (ob0, ob1)
        sin = (si0, si1)
        sout = (so0, so1)

        def start_in(ci, b):
            pltpu.async_copy(h_hbm.at[pl.ds(base + ci * C, C)], inb[b], sin[b])

        def start_out(ci, b):
            pltpu.async_copy(outb[b], out_hbm.at[pl.ds(base + ci * C, C)],
                             sout[b])

        def wait_in(b):
            pltpu.make_async_copy(
                h_hbm.at[pl.ds(base, C)], inb[b], sin[b]).wait()

        def wait_out(b):
            pltpu.make_async_copy(
                outb[b], out_hbm.at[pl.ds(base, C)], sout[b]).wait()

        def compute(b):
            src, dst = inb[b], outb[b]

            @plsc.parallel_loop(0, n_groups, unroll=4)
            def _group(g):
                col = g * L
                idxg = idx_v[pl.ds(col, L)]
                for r in range(R):
                    dst[pl.ds(r * NZ + col, L)] = plsc.load_gather(
                        src, [idxg + (r * NZ)])

        start_in(0, 0)
        start_in(1, 1)

        @pl.loop(0, n_chunks, step=2)
        def _chunk(ci):
            for b in range(2):
                cb = ci + b
                wait_in(b)

                @pl.when(cb >= 2)
                def _():
                    wait_out(b)

                compute(b)
                start_out(cb, b)

                @pl.when(cb + 2 < n_chunks)
                def _():
                    start_in(cb + 2, b)

        wait_out(0)
        wait_out(1)

    return k


def kernel(h, indices):
    B, NZ = h.shape
    info = plsc.get_sparse_core_info()
    k = _build(B, NZ, info.num_cores, info.num_subcores, R=8)
    return k(h.reshape(B * NZ), indices).reshape(B, NZ)


# speedup vs baseline: 6.0003x; 2.9383x over previous
import functools

import jax
import jax.numpy as jnp
from jax import lax
from jax.experimental import pallas as pl
from jax.experimental.pallas import tpu as pltpu
from jax.experimental.pallas import tpu_sc as plsc

L = 16


def _build(B, NZ, NC, NS, R):
    NW = NC * NS
    rows_per_w = B // NW
    n_chunks = rows_per_w // R
    n_groups = NZ // L
    mesh = plsc.VectorSubcoreMesh(core_axis_name="c", subcore_axis_name="s")

    @functools.partial(
        pl.kernel,
        out_type=jax.ShapeDtypeStruct((B, NZ), jnp.float32),
        mesh=mesh,
        compiler_params=pltpu.CompilerParams(needs_layout_passes=False),
        scratch_types=[
            pltpu.VMEM((NZ,), jnp.int32),
            pltpu.VMEM((R, NZ), jnp.float32),
            pltpu.VMEM((R, NZ), jnp.float32),
            pltpu.VMEM((R, NZ), jnp.float32),
            pltpu.VMEM((R, NZ), jnp.float32),
            pltpu.SemaphoreType.DMA,
            pltpu.SemaphoreType.DMA,
            pltpu.SemaphoreType.DMA,
            pltpu.SemaphoreType.DMA,
        ],
    )
    def k(h_hbm, idx_hbm, out_hbm, idx_v, in0, in1, ob0, ob1,
          si0, si1, so0, so1):
        wid = lax.axis_index("s") * NC + lax.axis_index("c")
        base = wid * rows_per_w
        pltpu.sync_copy(idx_hbm, idx_v)

        inb = (in0, in1)
        outb = (ob0, ob1)
        sin = (si0, si1)
        sout = (so0, so1)

        def start_in(ci, b):
            pltpu.async_copy(
                h_hbm.at[pl.ds(base + ci * R, R)], inb[b], sin[b])

        def start_out(ci, b):
            pltpu.async_copy(
                outb[b], out_hbm.at[pl.ds(base + ci * R, R)], sout[b])

        def wait_in(b):
            pltpu.make_async_copy(
                h_hbm.at[pl.ds(base, R)], inb[b], sin[b]).wait()

        def wait_out(b):
            pltpu.make_async_copy(
                outb[b], out_hbm.at[pl.ds(base, R)], sout[b]).wait()

        def compute(b):
            src, dst = inb[b], outb[b]

            @plsc.parallel_loop(0, n_groups, unroll=4)
            def _group(g):
                col = g * L
                idxg = idx_v[pl.ds(col, L)]
                for r in range(R):
                    rsel = jnp.full((L,), r, jnp.int32)
                    dst[r, pl.ds(col, L)] = plsc.load_gather(
                        src, [rsel, idxg])

        start_in(0, 0)
        start_in(1, 1)

        @pl.loop(0, n_chunks, step=2)
        def _chunk(ci):
            for b in range(2):
                cb = ci + b
                wait_in(b)

                @pl.when(cb >= 2)
                def _():
                    wait_out(b)

                compute(b)
                start_out(cb, b)

                @pl.when(cb + 2 < n_chunks)
                def _():
                    start_in(cb + 2, b)

        wait_out(0)
        wait_out(1)

    return k


def kernel(h, indices):
    B, NZ = h.shape
    info = plsc.get_sparse_core_info()
    k = _build(B, NZ, info.num_cores, info.num_subcores, R=8)
    return k(h, indices)
